# baseline (speedup 1.0000x reference)
import functools

import jax
import jax.numpy as jnp
from jax import lax
from jax.experimental import pallas as pl
from jax.experimental.pallas import tpu as pltpu
from jax.experimental.pallas import tpu_sc as plsc

NC = 2
NS = 16
L = 16
NW = NC * NS

B, T, D = 4, 2048, 1024
N = B * T
RPW = N // NW
SPW = T // NW
C = 16
NCH = RPW // C
PH = 2
PR = SPW // PH
HPH = PR // C
KD = D // L

DEPTH = 5
CHUNKS = [(b, sh * HPH + h2)
          for sh in range(PH) for b in range(B) for h2 in range(HPH)]


def _emb_body(x_hbm, tok_hbm, pos_hbm, out_hbm,
              idx_v, pos_v, tok0_v, tok1_v, tok2_v, tok3_v, tok4_v,
              sp, si, sg0, sg1, sg2, sg3, sg4, so0, so1, so2, so3, so4):
    c = lax.axis_index("c")
    s = lax.axis_index("s")
    wid = s * NC + c
    tok_v = (tok0_v, tok1_v, tok2_v, tok3_v, tok4_v)
    sg = (sg0, sg1, sg2, sg3, sg4)
    so = (so0, so1, so2, so3, so4)

    def stage_pos(sh):
        return pltpu.async_copy(
            pos_hbm.at[pl.ds(wid * SPW + sh * PR, PR)], pos_v, sp)

    pos_cp = stage_pos(0)
    idx_cps = [
        pltpu.async_copy(x_hbm.at[b, pl.ds(wid * SPW, SPW)],
                         idx_v.at[pl.ds(b * SPW, SPW)], si)
        for b in range(B)
    ]
    for cp in idx_cps:
        cp.wait()

    def gather(t, p):
        b, h = CHUNKS[t]
        return pltpu.async_copy(
            tok_hbm.at[idx_v.at[pl.ds(b * SPW + h * C, C)]], tok_v[p], sg[p])

    npc = B * HPH
    g = [None] * DEPTH
    o = [None] * DEPTH
    for t0 in range(DEPTH - 1):
        g[t0] = gather(t0, t0)
    for t in range(NCH):
        b, h = CHUNKS[t]
        p = t % DEPTH
        nt = t + DEPTH - 1
        if nt < NCH:
            q = nt % DEPTH
            if o[q] is not None:
                o[q].wait()
                o[q] = None
            g[q] = gather(nt, q)
        g[p].wait()
        if t % npc == 0:
            pos_cp.wait()

        buf = tok_v[p]
        h2 = h % HPH

        @plsc.parallel_loop(0, C * KD, 1, unroll=8)
        def add_body(i):
            r = i // KD
            k = (i % KD) * L
            buf[r, pl.ds(k, L)] = (buf[r, pl.ds(k, L)]
                                   + pos_v[h2 * C + r, pl.ds(k, L)])

        if t + 1 == npc:
            pos_cp = stage_pos(1)
        o[p] = pltpu.async_copy(
            buf, out_hbm.at[b, pl.ds(wid * SPW + h * C, C)], so[p])
    for cp in o:
        if cp is not None:
            cp.wait()


@jax.jit
def _emb(x, tok_table, pos_table):
    mesh = plsc.VectorSubcoreMesh(
        core_axis_name="c", subcore_axis_name="s",
        num_cores=NC, num_subcores=NS)
    return pl.kernel(
        _emb_body,
        out_type=jax.ShapeDtypeStruct((B, T, D), jnp.float32),
        mesh=mesh,
        scratch_types=[
            pltpu.VMEM((RPW,), jnp.int32),
            pltpu.VMEM((PR, D), jnp.float32),
        ] + [pltpu.VMEM((C, D), jnp.float32)] * DEPTH
          + [pltpu.SemaphoreType.DMA] * (2 + 2 * DEPTH),
    )(x, tok_table, pos_table)


def kernel(x, tok_table, pos_table):
    return _emb(x.astype(jnp.int32), tok_table, pos_table)


# speedup vs baseline: 1.0179x; 1.0179x over previous
import functools

import jax
import jax.numpy as jnp
from jax import lax
from jax.experimental import pallas as pl
from jax.experimental.pallas import tpu as pltpu
from jax.experimental.pallas import tpu_sc as plsc

NC = 2
NS = 16
L = 16
NW = NC * NS

B, T, D = 4, 2048, 1024
N = B * T
RPW = N // NW
SPW = T // NW
C = 16
NCH = RPW // C
HPS = SPW // C
KD = D // L


DEPTH = 3


def _emb_body(x_hbm, tok_hbm, pos_hbm, out_hbm,
              idx_v, pos_v, tok0_v, tok1_v, tok2_v,
              sp, si, sg0, sg1, sg2, so0, so1, so2):
    c = lax.axis_index("c")
    s = lax.axis_index("s")
    wid = s * NC + c
    tok_v = (tok0_v, tok1_v, tok2_v)
    sg = (sg0, sg1, sg2)
    so = (so0, so1, so2)

    pos_cp = pltpu.async_copy(pos_hbm.at[pl.ds(wid * SPW, SPW)], pos_v, sp)
    idx_cps = [
        pltpu.async_copy(x_hbm.at[b, pl.ds(wid * SPW, SPW)],
                         idx_v.at[pl.ds(b * SPW, SPW)], si)
        for b in range(B)
    ]
    for cp in idx_cps:
        cp.wait()

    def gather(t, p):
        return pltpu.async_copy(
            tok_hbm.at[idx_v.at[pl.ds(t * C, C)]], tok_v[p], sg[p])

    g = [None] * DEPTH
    o = [None] * DEPTH
    for t0 in range(DEPTH - 1):
        g[t0] = gather(t0, t0)
    for t in range(NCH):
        p = t % DEPTH
        nt = t + DEPTH - 1
        if nt < NCH:
            q = nt % DEPTH
            if o[q] is not None:
                o[q].wait()
                o[q] = None
            g[q] = gather(nt, q)
        g[p].wait()
        if t == 0:
            pos_cp.wait()

        b, h = t // HPS, t % HPS
        buf = tok_v[p]

        @plsc.parallel_loop(0, C * KD, 1, unroll=8)
        def add_body(i):
            r = i // KD
            k = (i % KD) * L
            plsc.addupdate(buf.at[r, pl.ds(k, L)],
                           pos_v[h * C + r, pl.ds(k, L)])

        o[p] = pltpu.async_copy(
            buf, out_hbm.at[b, pl.ds(wid * SPW + h * C, C)], so[p])
    for cp in o:
        if cp is not None:
            cp.wait()


@jax.jit
def _emb(x, tok_table, pos_table):
    mesh = plsc.VectorSubcoreMesh(
        core_axis_name="c", subcore_axis_name="s",
        num_cores=NC, num_subcores=NS)
    return pl.kernel(
        _emb_body,
        out_type=jax.ShapeDtypeStruct((B, T, D), jnp.float32),
        mesh=mesh,
        scratch_types=[
            pltpu.VMEM((RPW,), jnp.int32),
            pltpu.VMEM((SPW, D), jnp.float32),
            pltpu.VMEM((C, D), jnp.float32),
            pltpu.VMEM((C, D), jnp.float32),
            pltpu.VMEM((C, D), jnp.float32),
        ] + [pltpu.SemaphoreType.DMA] * (2 + 2 * DEPTH),
    )(x, tok_table, pos_table)


def kernel(x, tok_table, pos_table):
    return _emb(x.astype(jnp.int32), tok_table, pos_table)


# speedup vs baseline: 1.0796x; 1.0606x over previous
import functools

import jax
import jax.numpy as jnp
from jax import lax
from jax.experimental import pallas as pl
from jax.experimental.pallas import tpu as pltpu
from jax.experimental.pallas import tpu_sc as plsc

NC = 2
NS = 16
L = 16
NW = NC * NS

B, T, D = 4, 2048, 1024
N = B * T
RPW = N // NW
SPW = T // NW
C = 16
NCH = RPW // C
HPS = SPW // C
KD = D // L


DEPTH = 3
LA = 1


def _emb_body(x_hbm, tok_hbm, pos_hbm, out_hbm,
              idx_v, pos_v, tok0_v, tok1_v, tok2_v,
              sp, si, sg0, sg1, sg2, so0, so1, so2):
    c = lax.axis_index("c")
    s = lax.axis_index("s")
    wid = s * NC + c
    tok_v = (tok0_v, tok1_v, tok2_v)
    sg = (sg0, sg1, sg2)
    so = (so0, so1, so2)

    pos_cp = pltpu.async_copy(pos_hbm.at[pl.ds(wid * SPW, SPW)], pos_v, sp)
    idx_cps = [
        pltpu.async_copy(x_hbm.at[b, pl.ds(wid * SPW, SPW)],
                         idx_v.at[pl.ds(b * SPW, SPW)], si)
        for b in range(B)
    ]
    for cp in idx_cps:
        cp.wait()

    def gather(t, p):
        return pltpu.async_copy(
            tok_hbm.at[idx_v.at[pl.ds(t * C, C)]], tok_v[p], sg[p])

    g = [None] * DEPTH
    o = [None] * DEPTH
    for t0 in range(LA):
        g[t0] = gather(t0, t0)
    for t in range(NCH):
        p = t % DEPTH
        nt = t + LA
        if nt < NCH:
            q = nt % DEPTH
            if o[q] is not None:
                o[q].wait()
                o[q] = None
            g[q] = gather(nt, q)
        g[p].wait()
        if t == 0:
            pos_cp.wait()

        b, h = t // HPS, t % HPS
        buf = tok_v[p]

        @plsc.parallel_loop(0, C * KD, 1, unroll=8)
        def add_body(i):
            r = i // KD
            k = (i % KD) * L
            plsc.addupdate(buf.at[r, pl.ds(k, L)],
                           pos_v[h * C + r, pl.ds(k, L)])

        o[p] = pltpu.async_copy(
            buf, out_hbm.at[b, pl.ds(wid * SPW + h * C, C)], so[p])
    for cp in o:
        if cp is not None:
            cp.wait()


@jax.jit
def _emb(x, tok_table, pos_table):
    mesh = plsc.VectorSubcoreMesh(
        core_axis_name="c", subcore_axis_name="s",
        num_cores=NC, num_subcores=NS)
    return pl.kernel(
        _emb_body,
        out_type=jax.ShapeDtypeStruct((B, T, D), jnp.float32),
        mesh=mesh,
        scratch_types=[
            pltpu.VMEM((RPW,), jnp.int32),
            pltpu.VMEM((SPW, D), jnp.float32),
            pltpu.VMEM((C, D), jnp.float32),
            pltpu.VMEM((C, D), jnp.float32),
            pltpu.VMEM((C, D), jnp.float32),
        ] + [pltpu.SemaphoreType.DMA] * (2 + 2 * DEPTH),
    )(x, tok_table, pos_table)


def kernel(x, tok_table, pos_table):
    return _emb(x.astype(jnp.int32), tok_table, pos_table)


# speedup vs baseline: 1.0899x; 1.0095x over previous
import functools

import jax
import jax.numpy as jnp
from jax import lax
from jax.experimental import pallas as pl
from jax.experimental.pallas import tpu as pltpu
from jax.experimental.pallas import tpu_sc as plsc

NC = 2
NS = 16
L = 16
NW = NC * NS

B, T, D = 4, 2048, 1024
N = B * T
RPW = N // NW
SPW = T // NW
C = 16
NCH = RPW // C
PH = 2
PR = SPW // PH
HPH = PR // C
KD = D // L

DEPTH = 5
LA = 2
CHUNKS = [(b, sh * HPH + h2)
          for sh in range(PH) for b in range(B) for h2 in range(HPH)]


def _emb_body(x_hbm, tok_hbm, pos_hbm, out_hbm,
              idx_v, pos_v, tok0_v, tok1_v, tok2_v, tok3_v, tok4_v,
              sp, si, sg0, sg1, sg2, sg3, sg4, so0, so1, so2, so3, so4):
    c = lax.axis_index("c")
    s = lax.axis_index("s")
    wid = s * NC + c
    tok_v = (tok0_v, tok1_v, tok2_v, tok3_v, tok4_v)
    sg = (sg0, sg1, sg2, sg3, sg4)
    so = (so0, so1, so2, so3, so4)

    def stage_pos(sh):
        return pltpu.async_copy(
            pos_hbm.at[pl.ds(wid * SPW + sh * PR, PR)], pos_v, sp)

    pos_cp = stage_pos(0)
    idx_cps = [
        pltpu.async_copy(x_hbm.at[b, pl.ds(wid * SPW, SPW)],
                         idx_v.at[pl.ds(b * SPW, SPW)], si)
        for b in range(B)
    ]
    for cp in idx_cps:
        cp.wait()

    def gather(t, p):
        b, h = CHUNKS[t]
        return pltpu.async_copy(
            tok_hbm.at[idx_v.at[pl.ds(b * SPW + h * C, C)]], tok_v[p], sg[p])

    npc = B * HPH
    g = [None] * DEPTH
    o = [None] * DEPTH
    for t0 in range(LA):
        g[t0] = gather(t0, t0)
    for t in range(NCH):
        b, h = CHUNKS[t]
        p = t % DEPTH
        nt = t + LA
        if nt < NCH:
            q = nt % DEPTH
            if o[q] is not None:
                o[q].wait()
                o[q] = None
            g[q] = gather(nt, q)
        g[p].wait()
        if t % npc == 0:
            pos_cp.wait()

        buf = tok_v[p]
        h2 = h % HPH

        @plsc.parallel_loop(0, C * KD, 1, unroll=8)
        def add_body(i):
            r = i // KD
            k = (i % KD) * L
            plsc.addupdate(buf.at[r, pl.ds(k, L)],
                           pos_v[h2 * C + r, pl.ds(k, L)])

        if t + 1 == npc:
            pos_cp = stage_pos(1)
        o[p] = pltpu.async_copy(
            buf, out_hbm.at[b, pl.ds(wid * SPW + h * C, C)], so[p])
    for cp in o:
        if cp is not None:
            cp.wait()


@jax.jit
def _emb(x, tok_table, pos_table):
    mesh = plsc.VectorSubcoreMesh(
        core_axis_name="c", subcore_axis_name="s",
        num_cores=NC, num_subcores=NS)
    return pl.kernel(
        _emb_body,
        out_type=jax.ShapeDtypeStruct((B, T, D), jnp.float32),
        mesh=mesh,
        scratch_types=[
            pltpu.VMEM((RPW,), jnp.int32),
            pltpu.VMEM((PR, D), jnp.float32),
        ] + [pltpu.VMEM((C, D), jnp.float32)] * DEPTH
          + [pltpu.SemaphoreType.DMA] * (2 + 2 * DEPTH),
    )(x, tok_table, pos_table)


def kernel(x, tok_table, pos_table):
    return _emb(x.astype(jnp.int32), tok_table, pos_table)
